# baseline (speedup 1.0000x reference)
import functools

import jax
import jax.numpy as jnp
from jax import lax
from jax.experimental import pallas as pl
from jax.experimental.pallas import tpu as pltpu
from jax.experimental.pallas import tpu_sc as plsc

E = 64
CAP = 40
D = 768
DFF = 2048
NB = 2
S = 2048
T = NB * S
ECAP = NB * CAP
TRASH = E * ECAP
ROWS = 512
NBLK = T // ROWS
NW = 32
TPW = T // NW


def _router_body(x_ref, w_ref, logits_ref, dest_ref, eidx_ref, prob_ref,
                 pf0_ref, counts_ref):
    i = pl.program_id(0)
    x = x_ref[...]
    logits = jnp.dot(x, w_ref[...], preferred_element_type=jnp.float32)
    logits_ref[...] = logits
    m = jnp.max(logits, axis=1, keepdims=True)
    ex = jnp.exp(logits - m)
    probs = ex / jnp.sum(ex, axis=1, keepdims=True)
    pm = jnp.max(probs, axis=1, keepdims=True)
    prob_ref[...] = pm
    lane = lax.broadcasted_iota(jnp.int32, probs.shape, 1)
    top1 = jnp.min(jnp.where(probs == pm, lane, E), axis=1, keepdims=True)
    onehot = (lane == top1).astype(jnp.float32)
    r = lax.broadcasted_iota(jnp.int32, (ROWS, ROWS), 0)
    c = lax.broadcasted_iota(jnp.int32, (ROWS, ROWS), 1)
    tri = (c <= r).astype(jnp.float32)

    @pl.when((i == 0) | (i == NBLK // 2))
    def _():
        counts_ref[...] = jnp.zeros_like(counts_ref)

    cum = jnp.dot(tri, onehot, preferred_element_type=jnp.float32)
    cum = cum + counts_ref[...]
    counts_ref[...] = jnp.max(cum, axis=0, keepdims=True)
    rank = jnp.sum(onehot * cum, axis=1, keepdims=True)
    kept = rank <= CAP
    b = i // (NBLK // 2)
    slot = top1 * ECAP + b * CAP + rank.astype(jnp.int32) - 1
    dest_ref[...] = jnp.where(kept, slot, TRASH)
    eidx_ref[...] = jnp.where(kept, top1, 0)
    pf0_ref[...] = jnp.where(kept, 0.0, pm) * x


_router_call = pl.pallas_call(
    _router_body,
    grid=(NBLK,),
    in_specs=[
        pl.BlockSpec((ROWS, D), lambda i: (i, 0)),
        pl.BlockSpec((D, E), lambda i: (0, 0)),
    ],
    out_specs=[
        pl.BlockSpec((ROWS, E), lambda i: (i, 0)),
        pl.BlockSpec((ROWS, 1), lambda i: (i, 0)),
        pl.BlockSpec((ROWS, 1), lambda i: (i, 0)),
        pl.BlockSpec((ROWS, 1), lambda i: (i, 0)),
        pl.BlockSpec((ROWS, D), lambda i: (i, 0)),
    ],
    out_shape=[
        jax.ShapeDtypeStruct((T, E), jnp.float32),
        jax.ShapeDtypeStruct((T, 1), jnp.int32),
        jax.ShapeDtypeStruct((T, 1), jnp.int32),
        jax.ShapeDtypeStruct((T, 1), jnp.float32),
        jax.ShapeDtypeStruct((T, D), jnp.float32),
    ],
    scratch_shapes=[pltpu.VMEM((1, E), jnp.float32)],
)


HTPW = TPW // 2


def _expert_body(dest_ref, flat_ref, prob_ref, wi_ref, wo_ref, y_ref):
    e = pl.program_id(0)
    slots = e * ECAP + lax.broadcasted_iota(jnp.int32, (ECAP, 1), 0)
    g = (dest_ref[...] == slots).astype(jnp.float32)
    x = jnp.dot(g, flat_ref[...], preferred_element_type=jnp.float32)
    p = jnp.dot(g, prob_ref[...], preferred_element_type=jnp.float32)
    h = jnp.maximum(
        jnp.dot(x, wi_ref[0], preferred_element_type=jnp.float32), 0.0)
    y = jnp.dot(h, wo_ref[0], preferred_element_type=jnp.float32)

    @pl.when(e < E)
    def _():
        y_ref[...] = y * p

    @pl.when(e == E)
    def _():
        y_ref[...] = jnp.zeros_like(y_ref)


_expert_call = pl.pallas_call(
    _expert_body,
    grid=(E + 1,),
    in_specs=[
        pl.BlockSpec((1, T), lambda e: (0, 0)),
        pl.BlockSpec((T, D), lambda e: (0, 0)),
        pl.BlockSpec((T, 1), lambda e: (0, 0)),
        pl.BlockSpec((1, D, DFF), lambda e: (jnp.minimum(e, E - 1), 0, 0)),
        pl.BlockSpec((1, DFF, D), lambda e: (jnp.minimum(e, E - 1), 0, 0)),
    ],
    out_specs=pl.BlockSpec((ECAP, D), lambda e: (e, 0)),
    out_shape=jax.ShapeDtypeStruct((TRASH + 1, D), jnp.float32),
)


CH = 32
NCH = TPW // CH


def _gather_body(ydisp_hbm, dest_hbm, pf0_hbm, out_hbm,
                 idx_v, ybuf, pfbuf, sg0, sg1, sp0, sp1, sw0, sw1):
    wid = lax.axis_index("s") * 2 + lax.axis_index("c")
    base = wid * TPW
    pltpu.sync_copy(dest_hbm.at[pl.ds(base, TPW)], idx_v)
    sg = (sg0, sg1)
    sp = (sp0, sp1)
    sw = (sw0, sw1)
    g = [None] * NCH
    p = [None] * NCH
    w = [None] * NCH

    def start(c):
        b = c % 2
        g[c] = pltpu.async_copy(
            ydisp_hbm.at[idx_v.at[pl.ds(c * CH, CH)]], ybuf.at[b], sg[b])
        p[c] = pltpu.async_copy(
            pf0_hbm.at[pl.ds(base + c * CH, CH)], pfbuf.at[b], sp[b])

    start(0)
    for c in range(NCH):
        b = c % 2
        if c + 1 < NCH:
            if c >= 1:
                w[c - 1].wait()
            start(c + 1)
        g[c].wait()
        p[c].wait()

        def row_add(i, _, b=b):
            for k in range(D // 16):
                ybuf[b, i, pl.ds(k * 16, 16)] = (
                    ybuf[b, i, pl.ds(k * 16, 16)]
                    + pfbuf[b, i, pl.ds(k * 16, 16)])
            return 0

        lax.fori_loop(0, CH, row_add, 0)
        w[c] = pltpu.async_copy(ybuf.at[b],
                                out_hbm.at[pl.ds(base + c * CH, CH)], sw[b])
    w[NCH - 1].wait()


@functools.cache
def _gather_call():
    return functools.partial(
        pl.kernel,
        out_type=jax.ShapeDtypeStruct((T, D), jnp.float32),
        mesh=plsc.VectorSubcoreMesh(core_axis_name="c",
                                    subcore_axis_name="s"),
        scratch_types=[
            pltpu.VMEM((TPW,), jnp.int32),
            pltpu.VMEM((2, CH, D), jnp.float32),
            pltpu.VMEM((2, CH, D), jnp.float32),
            pltpu.SemaphoreType.DMA,
            pltpu.SemaphoreType.DMA,
            pltpu.SemaphoreType.DMA,
            pltpu.SemaphoreType.DMA,
            pltpu.SemaphoreType.DMA,
            pltpu.SemaphoreType.DMA,
        ],
    )(_gather_body)


def kernel(hidden_states, W_router, Wi, Wo):
    flat = hidden_states.reshape(T, D)
    logits, dest, eidx, prob, pf0 = _router_call(flat, W_router)
    ydisp = _expert_call(dest.reshape(1, T), flat, prob, Wi, Wo)
    out_flat = _gather_call()(ydisp, dest.reshape(T), pf0)
    return (out_flat.reshape(NB, S, D),
            logits.reshape(NB, S, E),
            eidx.reshape(NB, S))


# speedup vs baseline: 1.0079x; 1.0079x over previous
import functools

import jax
import jax.numpy as jnp
from jax import lax
from jax.experimental import pallas as pl
from jax.experimental.pallas import tpu as pltpu
from jax.experimental.pallas import tpu_sc as plsc

E = 64
CAP = 40
D = 768
DFF = 2048
NB = 2
S = 2048
T = NB * S
ECAP = NB * CAP
TRASH = E * ECAP
ROWS = 512
NBLK = T // ROWS
NW = 32
TPW = T // NW
NCOPY = -(-T // ECAP)
BIG = TRASH + NCOPY * ECAP


def _router_body(x_ref, w_ref, logits_ref, dest_ref, eidx_ref, probrep_ref,
                 counts_ref):
    i = pl.program_id(0)
    x = x_ref[...]
    logits = jnp.dot(x, w_ref[...], preferred_element_type=jnp.float32)
    logits_ref[...] = logits
    m = jnp.max(logits, axis=1, keepdims=True)
    ex = jnp.exp(logits - m)
    probs = ex / jnp.sum(ex, axis=1, keepdims=True)
    pm = jnp.max(probs, axis=1, keepdims=True)
    lane = lax.broadcasted_iota(jnp.int32, probs.shape, 1)
    top1 = jnp.min(jnp.where(probs == pm, lane, E), axis=1, keepdims=True)
    onehot = (lane == top1).astype(jnp.float32)
    r = lax.broadcasted_iota(jnp.int32, (ROWS, ROWS), 0)
    c = lax.broadcasted_iota(jnp.int32, (ROWS, ROWS), 1)
    tri = (c <= r).astype(jnp.float32)

    @pl.when((i == 0) | (i == NBLK // 2))
    def _():
        counts_ref[...] = jnp.zeros_like(counts_ref)

    cum = jnp.dot(tri, onehot, preferred_element_type=jnp.float32)
    cum = cum + counts_ref[...]
    counts_ref[...] = jnp.max(cum, axis=0, keepdims=True)
    rank = jnp.sum(onehot * cum, axis=1, keepdims=True)
    kept = rank <= CAP
    b = i // (NBLK // 2)
    slot = top1 * ECAP + b * CAP + rank.astype(jnp.int32) - 1
    tg = i * ROWS + lax.broadcasted_iota(jnp.int32, (ROWS, 1), 0)
    pass_row = TRASH + jnp.where(tg < (NCOPY - 1) * ECAP,
                                 tg, tg + NCOPY * ECAP - T)
    dest_ref[...] = jnp.where(kept, slot, pass_row)
    eidx_ref[...] = jnp.where(kept, top1, 0)
    probrep_ref[...] = jnp.broadcast_to(pm, (ROWS, 16))


_router_call = pl.pallas_call(
    _router_body,
    grid=(NBLK,),
    in_specs=[
        pl.BlockSpec((ROWS, D), lambda i: (i, 0)),
        pl.BlockSpec((D, E), lambda i: (0, 0)),
    ],
    out_specs=[
        pl.BlockSpec((ROWS, E), lambda i: (i, 0)),
        pl.BlockSpec((ROWS, 1), lambda i: (i, 0)),
        pl.BlockSpec((ROWS, 1), lambda i: (i, 0)),
        pl.BlockSpec((ROWS, 16), lambda i: (i, 0)),
    ],
    out_shape=[
        jax.ShapeDtypeStruct((T, E), jnp.float32),
        jax.ShapeDtypeStruct((T, 1), jnp.int32),
        jax.ShapeDtypeStruct((T, 1), jnp.int32),
        jax.ShapeDtypeStruct((T, 16), jnp.float32),
    ],
    scratch_shapes=[pltpu.VMEM((1, E), jnp.float32)],
)


HTPW = TPW // 2


def _expert_body(dest_ref, flat_ref, wi_ref, wo_ref, y_ref):
    e = pl.program_id(0)

    @pl.when(e < E)
    def _():
        slots = e * ECAP + lax.broadcasted_iota(jnp.int32, (ECAP, 1), 0)
        g = (dest_ref[...] == slots).astype(jnp.float32)
        x = jnp.dot(g, flat_ref[...], preferred_element_type=jnp.float32)
        h = jnp.maximum(
            jnp.dot(x, wi_ref[0], preferred_element_type=jnp.float32), 0.0)
        y_ref[...] = jnp.dot(h, wo_ref[0],
                             preferred_element_type=jnp.float32)

    @pl.when(e >= E)
    def _():
        start = jnp.minimum((e - E) * ECAP, T - ECAP)
        y_ref[...] = flat_ref[pl.ds(start, ECAP), :]


_expert_call = pl.pallas_call(
    _expert_body,
    grid=(E + NCOPY,),
    in_specs=[
        pl.BlockSpec((1, T), lambda e: (0, 0)),
        pl.BlockSpec((T, D), lambda e: (0, 0)),
        pl.BlockSpec((1, D, DFF),
                     lambda e: (jnp.minimum(e, E - 1), 0, 0)),
        pl.BlockSpec((1, DFF, D),
                     lambda e: (jnp.minimum(e, E - 1), 0, 0)),
    ],
    out_specs=pl.BlockSpec((ECAP, D), lambda e: (e, 0)),
    out_shape=jax.ShapeDtypeStruct((BIG, D), jnp.float32),
)


def _gather_body(ybig_hbm, dest_hbm, prep_hbm, out_hbm,
                 idx_v, rows_v, prep_v, sem_i0, sem_i1, sem_o0, sem_o1):
    wid = lax.axis_index("s") * 2 + lax.axis_index("c")
    base = wid * TPW
    pltpu.sync_copy(dest_hbm.at[pl.ds(base, HTPW)], idx_v.at[0])
    pltpu.sync_copy(dest_hbm.at[pl.ds(base + HTPW, HTPW)], idx_v.at[1])
    pltpu.sync_copy(prep_hbm.at[pl.ds(base, HTPW)], prep_v.at[0])
    pltpu.sync_copy(prep_hbm.at[pl.ds(base + HTPW, HTPW)], prep_v.at[1])
    in0 = pltpu.async_copy(ybig_hbm.at[idx_v.at[0]], rows_v.at[0], sem_i0)
    in1 = pltpu.async_copy(ybig_hbm.at[idx_v.at[1]], rows_v.at[1], sem_i1)
    outs = []
    for b, ib, sem_o in ((0, in0, sem_o0), (1, in1, sem_o1)):
        ib.wait()

        def row_scale(i, _, b=b):
            pv = prep_v[b, i]
            for k in range(D // 16):
                rows_v[b, i, pl.ds(k * 16, 16)] = (
                    rows_v[b, i, pl.ds(k * 16, 16)] * pv)
            return 0

        lax.fori_loop(0, HTPW, row_scale, 0)
        outs.append(pltpu.async_copy(
            rows_v.at[b], out_hbm.at[pl.ds(base + b * HTPW, HTPW)], sem_o))
    for o in outs:
        o.wait()


@functools.cache
def _gather_call():
    return functools.partial(
        pl.kernel,
        out_type=jax.ShapeDtypeStruct((T, D), jnp.float32),
        mesh=plsc.VectorSubcoreMesh(core_axis_name="c",
                                    subcore_axis_name="s"),
        scratch_types=[
            pltpu.VMEM((2, HTPW), jnp.int32),
            pltpu.VMEM((2, HTPW, D), jnp.float32),
            pltpu.VMEM((2, HTPW, 16), jnp.float32),
            pltpu.SemaphoreType.DMA,
            pltpu.SemaphoreType.DMA,
            pltpu.SemaphoreType.DMA,
            pltpu.SemaphoreType.DMA,
        ],
    )(_gather_body)


def kernel(hidden_states, W_router, Wi, Wo):
    flat = hidden_states.reshape(T, D)
    logits, dest, eidx, probrep = _router_call(flat, W_router)
    ybig = _expert_call(dest.reshape(1, T), flat, Wi, Wo)
    out_flat = _gather_call()(ybig, dest.reshape(T), probrep)
    return (out_flat.reshape(NB, S, D),
            logits.reshape(NB, S, E),
            eidx.reshape(NB, S))
